# baseline (speedup 1.0000x reference)
import functools

import jax
import jax.numpy as jnp
from jax import lax
from jax.experimental import pallas as pl
from jax.experimental.pallas import tpu as pltpu
from jax.experimental.pallas import tpu_sc as plsc

N = 10000
E = 160000
H = 128
CL = 16
G3 = 3 * H

NC = 2
NS = 16
NW = NC * NS
EPW = E // NW
CH = 128
NFULL = EPW // CH
TAIL = EPW - NFULL * CH
RPT = 624
REM = N - NS * RPT

_mesh = plsc.VectorSubcoreMesh(core_axis_name="c", subcore_axis_name="s")


BF = jnp.bfloat16
NCHK = E // CH
CPW = NCHK // NW


MW = 96


def _add_rows_bf(a_v, b_v, nrows):
    def body(i, _):
        for u in range(MW // 16):
            j = u * 16
            a_v[i, pl.ds(j, 16)] = a_v[i, pl.ds(j, 16)] + b_v[i, pl.ds(j, 16)]
        return 0
    lax.fori_loop(0, nrows, body, 0)


@functools.partial(
    pl.kernel,
    out_type=jax.ShapeDtypeStruct((E, H), jnp.float32),
    mesh=_mesh,
    scratch_types=[
        pltpu.VMEM(((CPW + 1) * CH,), jnp.int32),
        pltpu.VMEM(((CPW + 1) * CH,), jnp.int32),
        pltpu.VMEM((CH, H), jnp.float32),
        pltpu.VMEM((CH, H), jnp.float32),
        pltpu.VMEM((CH, H), jnp.float32),
        pltpu.VMEM((CH, H), jnp.float32),
        pltpu.VMEM((CH, H), jnp.float32),
        pltpu.VMEM((CH, H), jnp.float32),
        pltpu.SemaphoreType.DMA,
        pltpu.SemaphoreType.DMA,
        pltpu.SemaphoreType.DMA,
        pltpu.SemaphoreType.DMA,
        pltpu.SemaphoreType.DMA,
        pltpu.SemaphoreType.DMA,
        pltpu.SemaphoreType.DMA,
        pltpu.SemaphoreType.DMA,
        pltpu.SemaphoreType.DMA,
    ],
)
def _edge_gather(p_hbm, q_hbm, src_hbm, dst_hbm, out_hbm,
                 s_all, d_all, a0, b0, a1, b1, a2, b2,
                 sa0, sb0, sa1, sb1, sa2, sb2, ss0, ss1, ss2):
    cid = lax.axis_index("c")
    wid = lax.axis_index("s") * NC + cid
    cbase = wid * CPW
    base = cbase * CH

    pltpu.sync_copy(src_hbm.at[pl.ds(base, CPW * CH)],
                    s_all.at[pl.ds(0, CPW * CH)])
    pltpu.sync_copy(dst_hbm.at[pl.ds(base, CPW * CH)],
                    d_all.at[pl.ds(0, CPW * CH)])

    @pl.when(wid < NC)
    def _():
        xoff = (NW * CPW + wid) * CH
        pltpu.sync_copy(src_hbm.at[pl.ds(xoff, CH)],
                        s_all.at[pl.ds(CPW * CH, CH)])
        pltpu.sync_copy(dst_hbm.at[pl.ds(xoff, CH)],
                        d_all.at[pl.ds(CPW * CH, CH)])

    def start_gather(c, av, bv, sema, semb):
        pltpu.async_copy(p_hbm.at[s_all.at[pl.ds(c * CH, CH)]], av, sema)
        pltpu.async_copy(q_hbm.at[d_all.at[pl.ds(c * CH, CH)]], bv, semb)

    def wait_gather(av, bv, sema, semb):
        pltpu.make_async_copy(p_hbm.at[pl.ds(0, CH)], av, sema).wait()
        pltpu.make_async_copy(q_hbm.at[pl.ds(0, CH)], bv, semb).wait()

    def start_store(c, av, sems):
        pltpu.async_copy(av, out_hbm.at[pl.ds((cbase + c) * CH, CH)], sems)

    def wait_store(av, sems):
        pltpu.make_async_copy(av, out_hbm.at[pl.ds(0, CH)], sems).wait()

    bufs = ((a0, b0, sa0, sb0, ss0),
            (a1, b1, sa1, sb1, ss1),
            (a2, b2, sa2, sb2, ss2))
    for t, (av, bv, sga, sgb, sst) in enumerate(bufs):
        start_gather(t, av, bv, sga, sgb)

    def triple(k, _):
        for t, (av, bv, sga, sgb, sst) in enumerate(bufs):
            wait_gather(av, bv, sga, sgb)
            _add_rows_bf(av, bv, CH)
            start_store(3 * k + t, av, sst)
        for t, (av, bv, sga, sgb, sst) in enumerate(bufs):
            @pl.when(3 * k + 3 + t <= CPW - 1)
            def _(av=av, bv=bv, sga=sga, sgb=sgb, sst=sst, t=t):
                wait_store(av, sst)
                start_gather(3 * k + 3 + t, av, bv, sga, sgb)
        return 0

    lax.fori_loop(0, CPW // 3, triple, 0)

    for t, (av, bv, sga, sgb, sst) in enumerate(bufs):
        wait_store(av, sst)

    @pl.when(wid < NC)
    def _():
        start_gather(CPW, a0, b0, sa0, sb0)
        wait_gather(a0, b0, sa0, sb0)
        _add_rows_bf(a0, b0, CH)
        pltpu.sync_copy(a0, out_hbm.at[pl.ds((NW * CPW + wid) * CH, CH)])


EPW1 = E // NS
NFULL1 = EPW1 // CH
TAIL1 = EPW1 - NFULL1 * CH
HALF = N // 2
ACC = HALF + NS
ZRT = 312
ZB = 104
VR = 16


NCH1 = NFULL1 + 1


@functools.partial(
    pl.kernel,
    out_type=jax.ShapeDtypeStruct((N, H), jnp.float32),
    mesh=_mesh,
    scratch_types=[
        pltpu.VMEM((EPW1,), jnp.int32),
        pltpu.VMEM((NCH1, CH), jnp.int32),
        pltpu.VMEM((CH, H), jnp.float32),
        pltpu.VMEM((CH, H), jnp.float32),
        pltpu.VMEM((ZB, H), jnp.float32),
        pltpu.VMEM_SHARED((ACC, H), jnp.float32),
        pltpu.SemaphoreType.DMA,
        pltpu.SemaphoreType.DMA,
    ],
)
def _edge_scatter(m2_hbm, dst_hbm, out_hbm, d_all, dj2, m0, m1, z_v,
                  shared, sm0, sm1):
    cid = lax.axis_index("c")
    sid = lax.axis_index("s")
    base = sid * EPW1
    lo = cid * HALF
    dump = HALF + sid

    pltpu.sync_copy(dst_hbm.at[pl.ds(base, EPW1)], d_all)

    def remap(k, _):
        i = k // (CH // VR)
        j = (k % (CH // VR)) * VR
        v = d_all[pl.ds(k * VR, VR)]
        m = v - lo
        ok = (m >= 0) & (m < HALF)
        dj2[i, pl.ds(j, VR)] = jnp.where(ok, m, dump)
        return 0
    lax.fori_loop(0, EPW1 // VR, remap, 0)
    for u in range(TAIL1 // VR, CH // VR):
        dj2[NCH1 - 1, pl.ds(u * VR, VR)] = jnp.full((VR,), HALF, jnp.int32)

    def zbody(k, _):
        i = k // (H // VR)
        j = (k % (H // VR)) * VR
        z_v[i, pl.ds(j, VR)] = jnp.zeros((VR,), jnp.float32)
        return 0
    lax.fori_loop(0, ZB * (H // VR), zbody, 0)

    for r in range(ZRT // ZB):
        pltpu.sync_copy(z_v, shared.at[pl.ds(sid * ZRT + r * ZB, ZB)])

    @pl.when(sid == NS - 1)
    def _():
        pltpu.sync_copy(z_v.at[pl.ds(0, ACC - NS * ZRT)],
                        shared.at[pl.ds(NS * ZRT, ACC - NS * ZRT)])

    plsc.subcore_barrier()

    def start_load(c, mv, sem):
        pltpu.async_copy(m2_hbm.at[pl.ds(base + c * CH, CH)], mv, sem)

    def wait_load(mv, sem):
        pltpu.make_async_copy(m2_hbm.at[pl.ds(0, CH)], mv, sem).wait()

    start_load(0, m0, sm0)

    def pair(k, _):
        start_load(2 * k + 1, m1, sm1)
        wait_load(m0, sm0)
        pltpu.sync_copy(m0, shared.at[dj2.at[2 * k]], add=True)

        @pl.when(2 * k + 2 <= NFULL1 - 1)
        def _():
            start_load(2 * k + 2, m0, sm0)
        wait_load(m1, sm1)
        pltpu.sync_copy(m1, shared.at[dj2.at[2 * k + 1]], add=True)
        return 0

    lax.fori_loop(0, NFULL1 // 2, pair, 0)

    pltpu.sync_copy(m2_hbm.at[pl.ds(base + NFULL1 * CH, TAIL1)],
                    m0.at[pl.ds(0, TAIL1)])
    pltpu.sync_copy(m0, shared.at[dj2.at[NCH1 - 1]], add=True)

    plsc.subcore_barrier()
    pltpu.sync_copy(shared.at[pl.ds(sid * ZRT, ZRT)],
                    out_hbm.at[pl.ds(lo + sid * ZRT, ZRT)])

    @pl.when(sid == NS - 1)
    def _():
        pltpu.sync_copy(shared.at[pl.ds(NS * ZRT, HALF - NS * ZRT)],
                        out_hbm.at[pl.ds(lo + NS * ZRT, HALF - NS * ZRT)])


def _matmul_body(a_ref, b_ref, o_ref):
    o_ref[:] = jnp.dot(a_ref[:], b_ref[:], preferred_element_type=jnp.float32)


def _fuse_w3(apre, wm):
    return pl.pallas_call(
        _matmul_body,
        out_shape=jax.ShapeDtypeStruct((H, G3), jnp.float32),
    )(apre, wm)


BN = 1000
BE = 3200


def _pq_body(h_ref, w1s_ref, w1d_ref, b1_ref, p_ref, q_ref):
    h = h_ref[:]
    p_ref[:] = jnp.dot(h, w1s_ref[:], preferred_element_type=jnp.float32)
    q_ref[:] = (jnp.dot(h, w1d_ref[:], preferred_element_type=jnp.float32)
                + b1_ref[:])


def _compute_pq(h, w1s, w1d, b1p):
    return pl.pallas_call(
        _pq_body,
        grid=(N // BN,),
        in_specs=[
            pl.BlockSpec((BN, H), lambda i: (i, 0)),
            pl.BlockSpec((H, H), lambda i: (0, 0)),
            pl.BlockSpec((H, H), lambda i: (0, 0)),
            pl.BlockSpec((1, H), lambda i: (0, 0)),
        ],
        out_specs=[
            pl.BlockSpec((BN, H), lambda i: (i, 0)),
            pl.BlockSpec((BN, H), lambda i: (i, 0)),
        ],
        out_shape=[
            jax.ShapeDtypeStruct((N, H), jnp.float32),
            jax.ShapeDtypeStruct((N, H), jnp.float32),
        ],
    )(h, w1s, w1d, b1p)


def _mid_body(m1_ref, w2_ref, b2_ref, m2_ref):
    m1 = jnp.maximum(m1_ref[:], 0.0)
    m2 = jnp.dot(m1, w2_ref[:], preferred_element_type=jnp.float32) + b2_ref[:]
    m2_ref[:] = jnp.maximum(m2, 0.0)


def _compute_mid(m1, w2p, b2p):
    return pl.pallas_call(
        _mid_body,
        grid=(E // BE,),
        in_specs=[
            pl.BlockSpec((BE, H), lambda i: (i, 0)),
            pl.BlockSpec((H, H), lambda i: (0, 0)),
            pl.BlockSpec((1, H), lambda i: (0, 0)),
        ],
        out_specs=pl.BlockSpec((BE, H), lambda i: (i, 0)),
        out_shape=jax.ShapeDtypeStruct((E, H), jnp.float32),
    )(m1, w2p, b2p)


def _gru_body(last, seg_ref, h_ref, cl_ref, a_ref, ux_ref, wc_ref,
              bx_ref, bh_ref, wo_ref, bo_ref, w1s_ref, w1d_ref, b1_ref,
              *out_refs):
    seg = seg_ref[:]
    h = h_ref[:]
    gx = (jnp.dot(seg, a_ref[:], preferred_element_type=jnp.float32)
          + jnp.dot(cl_ref[:], wc_ref[:], preferred_element_type=jnp.float32)
          + bx_ref[:])
    gh = jnp.dot(h, ux_ref[:], preferred_element_type=jnp.float32) + bh_ref[:]
    xz, xr, xh = gx[:, 0:H], gx[:, H:2 * H], gx[:, 2 * H:G3]
    hz, hr, hn = gh[:, 0:H], gh[:, H:2 * H], gh[:, 2 * H:G3]
    z = jax.nn.sigmoid(xz + hz)
    r = jax.nn.sigmoid(xr + hr)
    hh = jnp.tanh(xh + r * hn)
    hnew = z * h + (1.0 - z) * hh
    if last:
        logits = (jnp.dot(hnew, wo_ref[:], preferred_element_type=jnp.float32)
                  + bo_ref[:])
        logits = logits - jnp.max(logits, axis=-1, keepdims=True)
        ex = jnp.exp(logits)
        out_refs[0][:] = ex / jnp.sum(ex, axis=-1, keepdims=True)
    else:
        out_refs[0][:] = hnew
        out_refs[1][:] = jnp.dot(hnew, w1s_ref[:],
                                 preferred_element_type=jnp.float32)
        out_refs[2][:] = (jnp.dot(hnew, w1d_ref[:],
                                  preferred_element_type=jnp.float32)
                          + b1_ref[:])


def _compute_gru(last, seg, h, cluesp, a, ux, wcp, bxp, bhp, wop, bop,
                 w1s, w1d, b1p):
    nout = 1 if last else 3
    out_shape = [jax.ShapeDtypeStruct((N, H), jnp.float32)] * nout
    out_specs = [pl.BlockSpec((BN, H), lambda i: (i, 0))] * nout
    return pl.pallas_call(
        functools.partial(_gru_body, last),
        grid=(N // BN,),
        in_specs=[
            pl.BlockSpec((BN, H), lambda i: (i, 0)),
            pl.BlockSpec((BN, H), lambda i: (i, 0)),
            pl.BlockSpec((BN, H), lambda i: (i, 0)),
            pl.BlockSpec((H, G3), lambda i: (0, 0)),
            pl.BlockSpec((H, G3), lambda i: (0, 0)),
            pl.BlockSpec((H, G3), lambda i: (0, 0)),
            pl.BlockSpec((1, G3), lambda i: (0, 0)),
            pl.BlockSpec((1, G3), lambda i: (0, 0)),
            pl.BlockSpec((H, H), lambda i: (0, 0)),
            pl.BlockSpec((1, H), lambda i: (0, 0)),
            pl.BlockSpec((H, H), lambda i: (0, 0)),
            pl.BlockSpec((H, H), lambda i: (0, 0)),
            pl.BlockSpec((1, H), lambda i: (0, 0)),
        ],
        out_specs=out_specs,
        out_shape=out_shape,
    )(seg, h, cluesp, a, ux, wcp, bxp, bhp, wop, bop, w1s, w1d, b1p)


def kernel(hidden_state, clues_one_hot, edge_index, W1, b1, W2, b2, W3, b3,
           Wx, Ux, bx, bh, Wo, bo):
    f32 = jnp.float32
    src = edge_index[0]
    dst = edge_index[1]

    w1s = jnp.zeros((H, H), f32).at[:, :96].set(W1[:H])
    w1d = jnp.zeros((H, H), f32).at[:, :96].set(W1[H:])
    b1p = jnp.zeros((1, H), f32).at[0, :96].set(b1)
    w2p = jnp.zeros((H, H), f32).at[:96, :96].set(W2)
    b2p = jnp.zeros((1, H), f32).at[0, :96].set(b2).at[0, 96].set(1.0)
    apre = jnp.zeros((H, H), f32).at[:96].set(W3).at[96].set(b3)
    wm = Wx[CL:]
    a = _fuse_w3(apre, wm)
    wcp = jnp.zeros((H, G3), f32).at[:CL].set(Wx[:CL])
    cluesp = jnp.zeros((N, H), f32).at[:, :CL].set(clues_one_hot)
    bxp = bx.reshape(1, G3)
    bhp = bh.reshape(1, G3)
    wop = jnp.zeros((H, H), f32).at[:, :9].set(Wo)
    bop = jnp.full((1, H), -1e30, f32).at[0, :9].set(bo)

    h = hidden_state
    p, q = _compute_pq(h, w1s, w1d, b1p)
    for step in range(2):
        m1 = _edge_gather(p, q, src, dst)
        m2 = _compute_mid(m1, w2p, b2p)
        seg = _edge_scatter(m2, dst)
        last = step == 1
        outs = _compute_gru(last, seg, h, cluesp, a, Ux, wcp,
                            bxp, bhp, wop, bop, w1s, w1d, b1p)
        if last:
            (probs,) = outs
        else:
            h, p, q = outs

    return probs[:, :9]


# speedup vs baseline: 1.0220x; 1.0220x over previous
import functools

import jax
import jax.numpy as jnp
from jax import lax
from jax.experimental import pallas as pl
from jax.experimental.pallas import tpu as pltpu
from jax.experimental.pallas import tpu_sc as plsc

N = 10000
E = 160000
H = 128
CL = 16
G3 = 3 * H

NC = 2
NS = 16
NW = NC * NS
EPW = E // NW
CH = 128
NFULL = EPW // CH
TAIL = EPW - NFULL * CH
RPT = 624
REM = N - NS * RPT

_mesh = plsc.VectorSubcoreMesh(core_axis_name="c", subcore_axis_name="s")


BF = jnp.bfloat16
NCHK = E // CH
CPW = NCHK // NW


MW = 96


def _add_rows_bf(a_v, b_v, nrows):
    def body(i, _):
        for u in range(MW // 16):
            j = u * 16
            a_v[i, pl.ds(j, 16)] = a_v[i, pl.ds(j, 16)] + b_v[i, pl.ds(j, 16)]
        return 0
    lax.fori_loop(0, nrows, body, 0)


@functools.partial(
    pl.kernel,
    out_type=jax.ShapeDtypeStruct((E, H), jnp.float32),
    mesh=_mesh,
    scratch_types=[
        pltpu.VMEM(((CPW + 1) * CH,), jnp.int32),
        pltpu.VMEM(((CPW + 1) * CH,), jnp.int32),
        pltpu.VMEM((CH, H), jnp.float32),
        pltpu.VMEM((CH, H), jnp.float32),
        pltpu.VMEM((CH, H), jnp.float32),
        pltpu.VMEM((CH, H), jnp.float32),
        pltpu.SemaphoreType.DMA,
        pltpu.SemaphoreType.DMA,
        pltpu.SemaphoreType.DMA,
        pltpu.SemaphoreType.DMA,
        pltpu.SemaphoreType.DMA,
        pltpu.SemaphoreType.DMA,
    ],
)
def _edge_gather(p_hbm, q_hbm, src_hbm, dst_hbm, out_hbm,
                 s_all, d_all, a0, b0, a1, b1,
                 sa0, sb0, sa1, sb1, ss0, ss1):
    cid = lax.axis_index("c")
    wid = lax.axis_index("s") * NC + cid
    cbase = wid * CPW
    base = cbase * CH

    pltpu.sync_copy(src_hbm.at[pl.ds(base, CPW * CH)],
                    s_all.at[pl.ds(0, CPW * CH)])
    pltpu.sync_copy(dst_hbm.at[pl.ds(base, CPW * CH)],
                    d_all.at[pl.ds(0, CPW * CH)])

    @pl.when(wid < NC)
    def _():
        xoff = (NW * CPW + wid) * CH
        pltpu.sync_copy(src_hbm.at[pl.ds(xoff, CH)],
                        s_all.at[pl.ds(CPW * CH, CH)])
        pltpu.sync_copy(dst_hbm.at[pl.ds(xoff, CH)],
                        d_all.at[pl.ds(CPW * CH, CH)])

    def start_gather(c, av, bv, sema, semb):
        pltpu.async_copy(p_hbm.at[s_all.at[pl.ds(c * CH, CH)]], av, sema)
        pltpu.async_copy(q_hbm.at[d_all.at[pl.ds(c * CH, CH)]], bv, semb)

    def wait_gather(av, bv, sema, semb):
        pltpu.make_async_copy(p_hbm.at[pl.ds(0, CH)], av, sema).wait()
        pltpu.make_async_copy(q_hbm.at[pl.ds(0, CH)], bv, semb).wait()

    def start_store(c, av, sems):
        pltpu.async_copy(av, out_hbm.at[pl.ds((cbase + c) * CH, CH)], sems)

    def wait_store(av, sems):
        pltpu.make_async_copy(av, out_hbm.at[pl.ds(0, CH)], sems).wait()

    start_gather(0, a0, b0, sa0, sb0)

    def pair(k, _):
        @pl.when(k > 0)
        def _():
            wait_store(a1, ss1)
        start_gather(2 * k + 1, a1, b1, sa1, sb1)
        wait_gather(a0, b0, sa0, sb0)
        _add_rows_bf(a0, b0, CH)
        start_store(2 * k, a0, ss0)
        wait_store(a0, ss0)

        @pl.when(2 * k + 2 <= CPW - 1)
        def _():
            start_gather(2 * k + 2, a0, b0, sa0, sb0)
        wait_gather(a1, b1, sa1, sb1)
        _add_rows_bf(a1, b1, CH)
        start_store(2 * k + 1, a1, ss1)
        return 0

    lax.fori_loop(0, CPW // 2, pair, 0)

    wait_store(a1, ss1)

    @pl.when(wid < NC)
    def _():
        start_gather(CPW, a1, b1, sa1, sb1)
    wait_gather(a0, b0, sa0, sb0)
    _add_rows_bf(a0, b0, CH)
    start_store(CPW - 1, a0, ss0)

    @pl.when(wid < NC)
    def _():
        wait_gather(a1, b1, sa1, sb1)
        _add_rows_bf(a1, b1, CH)
        pltpu.sync_copy(a1, out_hbm.at[pl.ds((NW * CPW + wid) * CH, CH)])
    wait_store(a0, ss0)


EPW1 = E // NS
NFULL1 = EPW1 // CH
TAIL1 = EPW1 - NFULL1 * CH
HALF = N // 2
ACC = HALF + NS
ZRT = 312
ZB = 104
VR = 16


NCH1 = NFULL1 + 1


@functools.partial(
    pl.kernel,
    out_type=jax.ShapeDtypeStruct((N, H), jnp.float32),
    mesh=_mesh,
    scratch_types=[
        pltpu.VMEM((EPW1,), jnp.int32),
        pltpu.VMEM((NCH1, CH), jnp.int32),
        pltpu.VMEM((CH, H), jnp.float32),
        pltpu.VMEM((CH, H), jnp.float32),
        pltpu.VMEM((ZB, H), jnp.float32),
        pltpu.VMEM_SHARED((ACC, H), jnp.float32),
        pltpu.SemaphoreType.DMA,
        pltpu.SemaphoreType.DMA,
    ],
)
def _edge_scatter(m2_hbm, dst_hbm, out_hbm, d_all, dj2, m0, m1, z_v,
                  shared, sm0, sm1):
    cid = lax.axis_index("c")
    sid = lax.axis_index("s")
    base = sid * EPW1
    lo = cid * HALF
    dump = HALF + sid

    pltpu.sync_copy(dst_hbm.at[pl.ds(base, EPW1)], d_all)

    def remap(k, _):
        i = k // (CH // VR)
        j = (k % (CH // VR)) * VR
        v = d_all[pl.ds(k * VR, VR)]
        m = v - lo
        ok = (m >= 0) & (m < HALF)
        dj2[i, pl.ds(j, VR)] = jnp.where(ok, m, dump)
        return 0
    lax.fori_loop(0, EPW1 // VR, remap, 0)
    for u in range(TAIL1 // VR, CH // VR):
        dj2[NCH1 - 1, pl.ds(u * VR, VR)] = jnp.full((VR,), HALF, jnp.int32)

    def zbody(k, _):
        i = k // (H // VR)
        j = (k % (H // VR)) * VR
        z_v[i, pl.ds(j, VR)] = jnp.zeros((VR,), jnp.float32)
        return 0
    lax.fori_loop(0, ZB * (H // VR), zbody, 0)

    for r in range(ZRT // ZB):
        pltpu.sync_copy(z_v, shared.at[pl.ds(sid * ZRT + r * ZB, ZB)])

    @pl.when(sid == NS - 1)
    def _():
        pltpu.sync_copy(z_v.at[pl.ds(0, ACC - NS * ZRT)],
                        shared.at[pl.ds(NS * ZRT, ACC - NS * ZRT)])

    plsc.subcore_barrier()

    def start_load(c, mv, sem):
        pltpu.async_copy(m2_hbm.at[pl.ds(base + c * CH, CH)], mv, sem)

    def wait_load(mv, sem):
        pltpu.make_async_copy(m2_hbm.at[pl.ds(0, CH)], mv, sem).wait()

    start_load(0, m0, sm0)

    def pair(k, _):
        start_load(2 * k + 1, m1, sm1)
        wait_load(m0, sm0)
        pltpu.sync_copy(m0, shared.at[dj2.at[2 * k]], add=True)

        @pl.when(2 * k + 2 <= NFULL1 - 1)
        def _():
            start_load(2 * k + 2, m0, sm0)
        wait_load(m1, sm1)
        pltpu.sync_copy(m1, shared.at[dj2.at[2 * k + 1]], add=True)
        return 0

    lax.fori_loop(0, NFULL1 // 2, pair, 0)

    pltpu.sync_copy(m2_hbm.at[pl.ds(base + NFULL1 * CH, TAIL1)],
                    m0.at[pl.ds(0, TAIL1)])
    pltpu.sync_copy(m0, shared.at[dj2.at[NCH1 - 1]], add=True)

    plsc.subcore_barrier()
    pltpu.sync_copy(shared.at[pl.ds(sid * ZRT, ZRT)],
                    out_hbm.at[pl.ds(lo + sid * ZRT, ZRT)])

    @pl.when(sid == NS - 1)
    def _():
        pltpu.sync_copy(shared.at[pl.ds(NS * ZRT, HALF - NS * ZRT)],
                        out_hbm.at[pl.ds(lo + NS * ZRT, HALF - NS * ZRT)])


def _matmul_body(a_ref, b_ref, o_ref):
    o_ref[:] = jnp.dot(a_ref[:], b_ref[:], preferred_element_type=jnp.float32)


def _fuse_w3(apre, wm):
    return pl.pallas_call(
        _matmul_body,
        out_shape=jax.ShapeDtypeStruct((H, G3), jnp.float32),
    )(apre, wm)


BN = 1000
BE = 3200


def _pq_body(h_ref, w1s_ref, w1d_ref, b1_ref, p_ref, q_ref):
    h = h_ref[:]
    p_ref[:] = jnp.dot(h, w1s_ref[:], preferred_element_type=jnp.float32)
    q_ref[:] = (jnp.dot(h, w1d_ref[:], preferred_element_type=jnp.float32)
                + b1_ref[:])


def _compute_pq(h, w1s, w1d, b1p):
    return pl.pallas_call(
        _pq_body,
        grid=(N // BN,),
        in_specs=[
            pl.BlockSpec((BN, H), lambda i: (i, 0)),
            pl.BlockSpec((H, H), lambda i: (0, 0)),
            pl.BlockSpec((H, H), lambda i: (0, 0)),
            pl.BlockSpec((1, H), lambda i: (0, 0)),
        ],
        out_specs=[
            pl.BlockSpec((BN, H), lambda i: (i, 0)),
            pl.BlockSpec((BN, H), lambda i: (i, 0)),
        ],
        out_shape=[
            jax.ShapeDtypeStruct((N, H), jnp.float32),
            jax.ShapeDtypeStruct((N, H), jnp.float32),
        ],
    )(h, w1s, w1d, b1p)


def _mid_body(m1_ref, w2_ref, b2_ref, m2_ref):
    m1 = jnp.maximum(m1_ref[:], 0.0)
    m2 = jnp.dot(m1, w2_ref[:], preferred_element_type=jnp.float32) + b2_ref[:]
    m2_ref[:] = jnp.maximum(m2, 0.0)


def _compute_mid(m1, w2p, b2p):
    return pl.pallas_call(
        _mid_body,
        grid=(E // BE,),
        in_specs=[
            pl.BlockSpec((BE, H), lambda i: (i, 0)),
            pl.BlockSpec((H, H), lambda i: (0, 0)),
            pl.BlockSpec((1, H), lambda i: (0, 0)),
        ],
        out_specs=pl.BlockSpec((BE, H), lambda i: (i, 0)),
        out_shape=jax.ShapeDtypeStruct((E, H), jnp.float32),
    )(m1, w2p, b2p)


def _gru_body(last, seg_ref, h_ref, cl_ref, a_ref, ux_ref, wc_ref,
              bx_ref, bh_ref, wo_ref, bo_ref, w1s_ref, w1d_ref, b1_ref,
              *out_refs):
    seg = seg_ref[:]
    h = h_ref[:]
    gx = (jnp.dot(seg, a_ref[:], preferred_element_type=jnp.float32)
          + jnp.dot(cl_ref[:], wc_ref[:], preferred_element_type=jnp.float32)
          + bx_ref[:])
    gh = jnp.dot(h, ux_ref[:], preferred_element_type=jnp.float32) + bh_ref[:]
    xz, xr, xh = gx[:, 0:H], gx[:, H:2 * H], gx[:, 2 * H:G3]
    hz, hr, hn = gh[:, 0:H], gh[:, H:2 * H], gh[:, 2 * H:G3]
    z = jax.nn.sigmoid(xz + hz)
    r = jax.nn.sigmoid(xr + hr)
    hh = jnp.tanh(xh + r * hn)
    hnew = z * h + (1.0 - z) * hh
    if last:
        logits = (jnp.dot(hnew, wo_ref[:], preferred_element_type=jnp.float32)
                  + bo_ref[:])
        logits = logits - jnp.max(logits, axis=-1, keepdims=True)
        ex = jnp.exp(logits)
        out_refs[0][:] = ex / jnp.sum(ex, axis=-1, keepdims=True)
    else:
        out_refs[0][:] = hnew
        out_refs[1][:] = jnp.dot(hnew, w1s_ref[:],
                                 preferred_element_type=jnp.float32)
        out_refs[2][:] = (jnp.dot(hnew, w1d_ref[:],
                                  preferred_element_type=jnp.float32)
                          + b1_ref[:])


def _compute_gru(last, seg, h, cluesp, a, ux, wcp, bxp, bhp, wop, bop,
                 w1s, w1d, b1p):
    nout = 1 if last else 3
    out_shape = [jax.ShapeDtypeStruct((N, H), jnp.float32)] * nout
    out_specs = [pl.BlockSpec((BN, H), lambda i: (i, 0))] * nout
    return pl.pallas_call(
        functools.partial(_gru_body, last),
        grid=(N // BN,),
        in_specs=[
            pl.BlockSpec((BN, H), lambda i: (i, 0)),
            pl.BlockSpec((BN, H), lambda i: (i, 0)),
            pl.BlockSpec((BN, H), lambda i: (i, 0)),
            pl.BlockSpec((H, G3), lambda i: (0, 0)),
            pl.BlockSpec((H, G3), lambda i: (0, 0)),
            pl.BlockSpec((H, G3), lambda i: (0, 0)),
            pl.BlockSpec((1, G3), lambda i: (0, 0)),
            pl.BlockSpec((1, G3), lambda i: (0, 0)),
            pl.BlockSpec((H, H), lambda i: (0, 0)),
            pl.BlockSpec((1, H), lambda i: (0, 0)),
            pl.BlockSpec((H, H), lambda i: (0, 0)),
            pl.BlockSpec((H, H), lambda i: (0, 0)),
            pl.BlockSpec((1, H), lambda i: (0, 0)),
        ],
        out_specs=out_specs,
        out_shape=out_shape,
    )(seg, h, cluesp, a, ux, wcp, bxp, bhp, wop, bop, w1s, w1d, b1p)


def kernel(hidden_state, clues_one_hot, edge_index, W1, b1, W2, b2, W3, b3,
           Wx, Ux, bx, bh, Wo, bo):
    f32 = jnp.float32
    src = edge_index[0]
    dst = edge_index[1]

    w1s = jnp.zeros((H, H), f32).at[:, :96].set(W1[:H])
    w1d = jnp.zeros((H, H), f32).at[:, :96].set(W1[H:])
    b1p = jnp.zeros((1, H), f32).at[0, :96].set(b1)
    w2p = jnp.zeros((H, H), f32).at[:96, :96].set(W2)
    b2p = jnp.zeros((1, H), f32).at[0, :96].set(b2).at[0, 96].set(1.0)
    apre = jnp.zeros((H, H), f32).at[:96].set(W3).at[96].set(b3)
    wm = Wx[CL:]
    a = _fuse_w3(apre, wm)
    wcp = jnp.zeros((H, G3), f32).at[:CL].set(Wx[:CL])
    cluesp = jnp.zeros((N, H), f32).at[:, :CL].set(clues_one_hot)
    bxp = bx.reshape(1, G3)
    bhp = bh.reshape(1, G3)
    wop = jnp.zeros((H, H), f32).at[:, :9].set(Wo)
    bop = jnp.full((1, H), -1e30, f32).at[0, :9].set(bo)

    h = hidden_state
    p, q = _compute_pq(h, w1s, w1d, b1p)
    for step in range(2):
        m1 = _edge_gather(p, q, src, dst)
        m2 = _compute_mid(m1, w2p, b2p)
        seg = _edge_scatter(m2, dst)
        last = step == 1
        outs = _compute_gru(last, seg, h, cluesp, a, Ux, wcp,
                            bxp, bhp, wop, bop, w1s, w1d, b1p)
        if last:
            (probs,) = outs
        else:
            h, p, q = outs

    return probs[:, :9]
